# baseline (speedup 1.0000x reference)
import functools

import jax
import jax.numpy as jnp
from jax.experimental import pallas as pl

B, K, D = 2048, 256, 128
BLOCK_B = 128


def _loss_kernel(x_ref, c_ref, out_ref):
    x = x_ref[...]
    i = pl.program_id(0)

    inf = jnp.float32(jnp.inf)
    min1 = jnp.full((BLOCK_B, D), inf, jnp.float32)
    min2 = jnp.full((BLOCK_B, D), inf, jnp.float32)
    arg1 = jnp.zeros((BLOCK_B, D), jnp.int32)
    arg2 = jnp.zeros((BLOCK_B, D), jnp.int32)

    def body(kk, carry):
        m1, a1, m2, a2 = carry
        k = K - 1 - kk
        crow = c_ref[pl.ds(k, 1), :]
        d = x - crow
        d = d * d
        le1 = d <= m1
        le2 = d <= m2
        ki = jnp.full((), k, jnp.int32)
        new_m2 = jnp.where(le1, m1, jnp.where(le2, d, m2))
        new_a2 = jnp.where(le1, a1, jnp.where(le2, ki, a2))
        new_m1 = jnp.where(le1, d, m1)
        new_a1 = jnp.where(le1, ki, a1)
        return new_m1, new_a1, new_m2, new_a2

    arg1, arg2 = jax.lax.fori_loop(
        0, K, body, (min1, arg1, min2, arg2))[1::2]

    args = jnp.concatenate([arg1, arg2], axis=0)
    counts = jnp.zeros((2 * BLOCK_B, K), jnp.int32)
    iota_k = jax.lax.broadcasted_iota(jnp.int32, (2 * BLOCK_B, K), 1)
    for dd in range(D):
        col = jax.lax.slice(args, (0, dd), (2 * BLOCK_B, dd + 1))
        counts = counts + (col == iota_k).astype(jnp.int32)
    key = counts * K + (K - 1 - iota_k)
    best = jnp.max(key, axis=1, keepdims=True)
    mode_idx = (K - 1) - (best % K)

    onehot = (jax.lax.broadcasted_iota(jnp.int32, (2 * BLOCK_B, K), 1)
              == mode_idx).astype(jnp.float32)
    sel = jax.lax.dot(onehot, c_ref[...],
                      preferred_element_type=jnp.float32)
    pos = jax.lax.slice(sel, (0, 0), (BLOCK_B, D))
    neg = jax.lax.slice(sel, (BLOCK_B, 0), (2 * BLOCK_B, D))

    eps = jnp.float32(1e-6)
    dap = x - pos + eps
    dan = x - neg + eps
    dpn = pos - neg + eps
    d_ap = jnp.sqrt(jnp.sum(dap * dap, axis=1, keepdims=True))
    d_an = jnp.sqrt(jnp.sum(dan * dan, axis=1, keepdims=True))
    d_pn = jnp.sqrt(jnp.sum(dpn * dpn, axis=1, keepdims=True))
    d_neg = jnp.minimum(d_an, d_pn)
    partial = jnp.sum(jnp.maximum(d_ap - d_neg + 1.0, 0.0),
                      axis=0, keepdims=True)

    @pl.when(i == 0)
    def _():
        out_ref[...] = jnp.zeros((1, 1), jnp.float32)

    out_ref[...] += partial


@jax.jit
def kernel(input_features, centroids):
    total = pl.pallas_call(
        _loss_kernel,
        grid=(B // BLOCK_B,),
        in_specs=[
            pl.BlockSpec((BLOCK_B, D), lambda i: (i, 0)),
            pl.BlockSpec((K, D), lambda i: (0, 0)),
        ],
        out_specs=pl.BlockSpec((1, 1), lambda i: (0, 0)),
        out_shape=jax.ShapeDtypeStruct((1, 1), jnp.float32),
    )(input_features, centroids)
    return total[0, 0] / B


# speedup vs baseline: 1.3487x; 1.3487x over previous
import functools

import jax
import jax.numpy as jnp
from jax.experimental import pallas as pl

B, K, D = 2048, 256, 128
BLOCK_B = 64


def _loss_kernel(x_ref, c_ref, out_ref):
    x = x_ref[...]
    i = pl.program_id(0)

    inf = jnp.float32(jnp.inf)
    min1 = jnp.full((BLOCK_B, D), inf, jnp.float32)
    min2 = jnp.full((BLOCK_B, D), inf, jnp.float32)
    arg1 = jnp.zeros((BLOCK_B, D), jnp.int32)
    arg2 = jnp.zeros((BLOCK_B, D), jnp.int32)

    def body(kk, carry):
        m1, a1, m2, a2 = carry
        k = K - 1 - kk
        crow = c_ref[pl.ds(k, 1), :]
        d = x - crow
        d = d * d
        le1 = d <= m1
        le2 = d <= m2
        ki = jnp.full((), k, jnp.int32)
        new_m2 = jnp.where(le1, m1, jnp.where(le2, d, m2))
        new_a2 = jnp.where(le1, a1, jnp.where(le2, ki, a2))
        new_m1 = jnp.where(le1, d, m1)
        new_a1 = jnp.where(le1, ki, a1)
        return new_m1, new_a1, new_m2, new_a2

    arg1, arg2 = jax.lax.fori_loop(
        0, K, body, (min1, arg1, min2, arg2))[1::2]

    args = jnp.concatenate([arg1, arg2], axis=0).astype(jnp.int16)
    counts = jnp.zeros((2 * BLOCK_B, K), jnp.int16)
    iota_k16 = jax.lax.broadcasted_iota(jnp.int16, (2 * BLOCK_B, K), 1)
    for dd in range(D):
        col = jax.lax.slice(args, (0, dd), (2 * BLOCK_B, dd + 1))
        counts = counts + (col == iota_k16).astype(jnp.int16)
    iota_k = jax.lax.broadcasted_iota(jnp.int32, (2 * BLOCK_B, K), 1)
    key = counts.astype(jnp.int32) * K + (K - 1 - iota_k)
    best = jnp.max(key, axis=1, keepdims=True)
    mode_idx = (K - 1) - (best % K)

    onehot = (jax.lax.broadcasted_iota(jnp.int32, (2 * BLOCK_B, K), 1)
              == mode_idx).astype(jnp.float32)
    sel = jax.lax.dot(onehot, c_ref[...],
                      preferred_element_type=jnp.float32)
    pos = jax.lax.slice(sel, (0, 0), (BLOCK_B, D))
    neg = jax.lax.slice(sel, (BLOCK_B, 0), (2 * BLOCK_B, D))

    eps = jnp.float32(1e-6)
    dap = x - pos + eps
    dan = x - neg + eps
    dpn = pos - neg + eps
    d_ap = jnp.sqrt(jnp.sum(dap * dap, axis=1, keepdims=True))
    d_an = jnp.sqrt(jnp.sum(dan * dan, axis=1, keepdims=True))
    d_pn = jnp.sqrt(jnp.sum(dpn * dpn, axis=1, keepdims=True))
    d_neg = jnp.minimum(d_an, d_pn)
    partial = jnp.sum(jnp.maximum(d_ap - d_neg + 1.0, 0.0),
                      axis=0, keepdims=True)

    @pl.when(i == 0)
    def _():
        out_ref[...] = jnp.zeros((1, 1), jnp.float32)

    out_ref[...] += partial


@jax.jit
def kernel(input_features, centroids):
    total = pl.pallas_call(
        _loss_kernel,
        grid=(B // BLOCK_B,),
        in_specs=[
            pl.BlockSpec((BLOCK_B, D), lambda i: (i, 0)),
            pl.BlockSpec((K, D), lambda i: (0, 0)),
        ],
        out_specs=pl.BlockSpec((1, 1), lambda i: (0, 0)),
        out_shape=jax.ShapeDtypeStruct((1, 1), jnp.float32),
    )(input_features, centroids)
    return total[0, 0] / B
